# baseline (speedup 1.0000x reference)
import functools

import jax
import jax.numpy as jnp
from jax import lax
from jax.experimental import pallas as pl
from jax.experimental.pallas import tpu as pltpu
from jax.experimental.pallas import tpu_sc as plsc

NC = 2
NS = 16
NW = NC * NS
EK = 128
DEGW = 16
ZB = 128


def _vsc_mesh():
    return plsc.VectorSubcoreMesh(core_axis_name="c", subcore_axis_name="s")


def _pad_up(v, m):
    return ((v + m - 1) // m) * m


def _sc_deg(np_, ep):
    rows = ep // EK
    rpt = rows // NW
    npt = np_ // NS

    @functools.partial(
        pl.kernel,
        mesh=_vsc_mesh(),
        out_type=jax.ShapeDtypeStruct((NC, np_, DEGW), jnp.float32),
        scratch_types=[
            pltpu.VMEM((rpt, EK), jnp.int32),
            pltpu.VMEM((EK, DEGW), jnp.float32),
            pltpu.VMEM((ZB, DEGW), jnp.float32),
            pltpu.VMEM_SHARED((np_, DEGW), jnp.float32),
        ],
    )
    def k(dst_hbm, out_hbm, idx_v, ones_v, zb_v, acc_sh):
        c = lax.axis_index("c")
        s = lax.axis_index("s")
        w = c * NS + s

        def fill(i, _):
            ones_v[i, :] = jnp.ones((DEGW,), jnp.float32)
            return 0

        lax.fori_loop(0, EK, fill, 0)

        def fillz(i, _):
            zb_v[i, :] = jnp.zeros((DEGW,), jnp.float32)
            return 0

        lax.fori_loop(0, ZB, fillz, 0)

        def zloop(b, _):
            pltpu.sync_copy(zb_v, acc_sh.at[pl.ds(s * npt + b * ZB, ZB)])
            return 0

        lax.fori_loop(0, npt // ZB, zloop, 0)
        plsc.subcore_barrier()

        pltpu.sync_copy(dst_hbm.at[pl.ds(w * rpt, rpt)], idx_v)

        def body(j, _):
            pltpu.sync_copy(ones_v, acc_sh.at[idx_v.at[j]], add=True)
            return 0

        lax.fori_loop(0, rpt, body, 0)
        plsc.subcore_barrier()

        def wloop(b, _):
            r0 = s * npt + b * ZB
            pltpu.sync_copy(acc_sh.at[pl.ds(r0, ZB)], zb_v)
            pltpu.sync_copy(zb_v, out_hbm.at[c, pl.ds(r0, ZB)])
            return 0

        lax.fori_loop(0, npt // ZB, wloop, 0)

    return k


def _sc_agg(n, np_, ep, d):
    rows = ep // EK
    rpt = rows // NW
    nph = 2
    cpp = rpt // nph
    npt = np_ // NS

    @functools.partial(
        pl.kernel,
        mesh=_vsc_mesh(),
        out_type=jax.ShapeDtypeStruct((NC, np_, d), jnp.float32),
        scratch_types=[
            pltpu.VMEM((cpp, EK), jnp.int32),
            pltpu.VMEM((cpp, EK), jnp.int32),
            pltpu.VMEM((2, EK, d), jnp.float32),
            pltpu.VMEM_SHARED((np_, d), jnp.float32),
            pltpu.SemaphoreType.DMA,
            pltpu.SemaphoreType.DMA,
        ],
    )
    def k(hs_hbm, src_hbm, dst_hbm, z_hbm, out_hbm, sidx, didx, rb, acc_sh,
          sem0, sem1):
        c = lax.axis_index("c")
        s = lax.axis_index("s")
        w = c * NS + s

        pltpu.sync_copy(z_hbm, rb.at[0])

        def zloop(b, _):
            pltpu.sync_copy(rb.at[0], acc_sh.at[pl.ds(s * npt + b * ZB, ZB)])
            return 0

        lax.fori_loop(0, npt // ZB, zloop, 0)
        plsc.subcore_barrier()

        for p in range(nph):
            r0 = w * rpt + p * cpp
            pltpu.sync_copy(src_hbm.at[pl.ds(r0, cpp)], sidx)
            pltpu.sync_copy(dst_hbm.at[pl.ds(r0, cpp)], didx)

            pltpu.async_copy(hs_hbm.at[sidx.at[0]], rb.at[0], sem0)

            def body(j, _):
                pltpu.async_copy(hs_hbm.at[sidx.at[2 * j + 1]], rb.at[1], sem1)
                pltpu.make_async_copy(
                    hs_hbm.at[sidx.at[2 * j]], rb.at[0], sem0).wait()
                pltpu.sync_copy(rb.at[0], acc_sh.at[didx.at[2 * j]], add=True)

                @pl.when(j < cpp // 2 - 1)
                def _():
                    pltpu.async_copy(
                        hs_hbm.at[sidx.at[2 * j + 2]], rb.at[0], sem0)

                pltpu.make_async_copy(
                    hs_hbm.at[sidx.at[2 * j + 1]], rb.at[1], sem1).wait()
                pltpu.sync_copy(rb.at[1], acc_sh.at[didx.at[2 * j + 1]],
                                add=True)
                return 0

            lax.fori_loop(0, cpp // 2, body, 0)

        plsc.subcore_barrier()

        def wloop(b, _):
            r0 = s * npt + b * ZB
            pltpu.sync_copy(acc_sh.at[pl.ds(r0, ZB)], rb.at[0])
            pltpu.sync_copy(rb.at[0], out_hbm.at[c, pl.ds(r0, ZB)])
            return 0

        lax.fori_loop(0, npt // ZB, wloop, 0)

    return k


def _dinv_from(degp_ref):
    deg = 1.0 + degp_ref[0, :, 0] + degp_ref[1, :, 0]
    return lax.rsqrt(deg)


def _tc_first(n, d, r):
    def body(x_ref, w_ref, degp_ref, hs_ref):
        dinv = _dinv_from(degp_ref)
        h = jnp.dot(x_ref[...], w_ref[...], preferred_element_type=jnp.float32)
        hs_ref[...] = h * dinv[:, None]

    return pl.pallas_call(
        body,
        grid=(n // r,),
        in_specs=[
            pl.BlockSpec((r, d), lambda i: (i, 0)),
            pl.BlockSpec((d, d), lambda i: (0, 0)),
            pl.BlockSpec((2, r, DEGW), lambda i: (0, i, 0)),
        ],
        out_specs=pl.BlockSpec((r, d), lambda i: (i, 0)),
        out_shape=jax.ShapeDtypeStruct((n, d), jnp.float32),
    )


def _tc_mid(n, d, r, relu):
    def body(aggp_ref, hsp_ref, b_ref, degp_ref, w_ref, out_ref):
        dinv = _dinv_from(degp_ref)
        t = (aggp_ref[0] + aggp_ref[1] + hsp_ref[...]) * dinv[:, None] + b_ref[...]
        if relu:
            t = jnp.maximum(t, 0.0)
        h = jnp.dot(t, w_ref[...], preferred_element_type=jnp.float32)
        out_ref[...] = h * dinv[:, None]

    return pl.pallas_call(
        body,
        grid=(n // r,),
        in_specs=[
            pl.BlockSpec((2, r, d), lambda i: (0, i, 0)),
            pl.BlockSpec((r, d), lambda i: (i, 0)),
            pl.BlockSpec((1, d), lambda i: (0, 0)),
            pl.BlockSpec((2, r, DEGW), lambda i: (0, i, 0)),
            pl.BlockSpec((d, d), lambda i: (0, 0)),
        ],
        out_specs=pl.BlockSpec((r, d), lambda i: (i, 0)),
        out_shape=jax.ShapeDtypeStruct((n, d), jnp.float32),
    )


def _tc_last(n, d, r):
    def body(aggp_ref, hsp_ref, degp_ref, out_ref):
        dinv = _dinv_from(degp_ref)
        out_ref[...] = (aggp_ref[0] + aggp_ref[1] + hsp_ref[...]) * dinv[:, None]

    return pl.pallas_call(
        body,
        grid=(n // r,),
        in_specs=[
            pl.BlockSpec((2, r, d), lambda i: (0, i, 0)),
            pl.BlockSpec((r, d), lambda i: (i, 0)),
            pl.BlockSpec((2, r, DEGW), lambda i: (0, i, 0)),
        ],
        out_specs=pl.BlockSpec((r, d), lambda i: (i, 0)),
        out_shape=jax.ShapeDtypeStruct((n, d), jnp.float32),
    )


def kernel(x, edge_index, W_in, b_in, W_h0, b_h0, W_h1, b_h1, W_out):
    n, d = x.shape
    e = edge_index.shape[1]
    r = 1000

    np_ = _pad_up(n + 1, NS * ZB)
    ep = _pad_up(e, NW * EK * 8)

    pad = ep - e
    srcp = jnp.concatenate(
        [edge_index[0], jnp.zeros((pad,), edge_index.dtype)]).reshape(ep // EK, EK)
    dstp = jnp.concatenate(
        [edge_index[1], jnp.full((pad,), n, edge_index.dtype)]).reshape(ep // EK, EK)

    zeros_rd = jnp.zeros((EK, d), jnp.float32)

    degp = _sc_deg(np_, ep)(dstp)
    agg_k = _sc_agg(n, np_, ep, d)

    def agg(hs, sp, dp):
        return agg_k(hs, sp, dp, zeros_rd)
    first = _tc_first(n, d, r)
    mid_nr = _tc_mid(n, d, r, relu=False)
    mid_re = _tc_mid(n, d, r, relu=True)
    last = _tc_last(n, d, r)

    b2_in = b_in.reshape(1, d)
    b2_h0 = b_h0.reshape(1, d)
    b2_h1 = b_h1.reshape(1, d)

    hs0 = first(x, W_in, degp)
    a0 = agg(hs0, srcp, dstp)
    hs1 = mid_nr(a0, hs0, b2_in, degp, W_h0)
    a1 = agg(hs1, srcp, dstp)
    hs2 = mid_re(a1, hs1, b2_h0, degp, W_h1)
    a2 = agg(hs2, srcp, dstp)
    hs3 = mid_re(a2, hs2, b2_h1, degp, W_out)
    a3 = agg(hs3, srcp, dstp)
    return last(a3, hs3, degp)


# speedup vs baseline: 3.7193x; 3.7193x over previous
import functools

import jax
import jax.numpy as jnp
from jax import lax
from jax.experimental import pallas as pl
from jax.experimental.pallas import tpu as pltpu
from jax.experimental.pallas import tpu_sc as plsc

NC = 2
NS = 16
NW = NC * NS
EK = 128
DEGW = 16
ZB = 128


def _vsc_mesh():
    return plsc.VectorSubcoreMesh(core_axis_name="c", subcore_axis_name="s")


def _pad_up(v, m):
    return ((v + m - 1) // m) * m


def _sc_deg(np_, ep):
    rows = ep // EK
    rpt = rows // NW
    npt = np_ // NS

    @functools.partial(
        pl.kernel,
        mesh=_vsc_mesh(),
        out_type=jax.ShapeDtypeStruct((NC, np_, DEGW), jnp.float32),
        scratch_types=[
            pltpu.VMEM((rpt, EK), jnp.int32),
            pltpu.VMEM((EK, DEGW), jnp.float32),
            pltpu.VMEM((ZB, DEGW), jnp.float32),
            pltpu.VMEM_SHARED((np_, DEGW), jnp.float32),
        ],
    )
    def k(dst_hbm, out_hbm, idx_v, ones_v, zb_v, acc_sh):
        c = lax.axis_index("c")
        s = lax.axis_index("s")
        w = c * NS + s

        def fill(i, _):
            ones_v[i, :] = jnp.ones((DEGW,), jnp.float32)
            return 0

        lax.fori_loop(0, EK, fill, 0)

        def fillz(i, _):
            zb_v[i, :] = jnp.zeros((DEGW,), jnp.float32)
            return 0

        lax.fori_loop(0, ZB, fillz, 0)

        def zloop(b, _):
            pltpu.sync_copy(zb_v, acc_sh.at[pl.ds(s * npt + b * ZB, ZB)])
            return 0

        lax.fori_loop(0, npt // ZB, zloop, 0)
        plsc.subcore_barrier()

        pltpu.sync_copy(dst_hbm.at[pl.ds(w * rpt, rpt)], idx_v)

        def body(j, _):
            pltpu.sync_copy(ones_v, acc_sh.at[idx_v.at[j]], add=True)
            return 0

        lax.fori_loop(0, rpt, body, 0)
        plsc.subcore_barrier()

        def wloop(b, _):
            r0 = s * npt + b * ZB
            pltpu.sync_copy(acc_sh.at[pl.ds(r0, ZB)], zb_v)
            pltpu.sync_copy(zb_v, out_hbm.at[c, pl.ds(r0, ZB)])
            return 0

        lax.fori_loop(0, npt // ZB, wloop, 0)

    return k


def _sc_agg(n, np_, ep, d, mode="full"):
    rows = ep // EK
    rpt = rows // NW
    nph = 2
    cpp = rpt // nph
    npt = np_ // NS

    @functools.partial(
        pl.kernel,
        mesh=_vsc_mesh(),
        out_type=jax.ShapeDtypeStruct((NC, np_, d), jnp.float32),
        scratch_types=[
            pltpu.VMEM((cpp, EK), jnp.int32),
            pltpu.VMEM((cpp, EK), jnp.int32),
            pltpu.VMEM((2, EK, d), jnp.float32),
            pltpu.VMEM_SHARED((np_, d), jnp.float32),
            pltpu.SemaphoreType.DMA,
            pltpu.SemaphoreType.DMA,
        ],
    )
    def k(hs_hbm, src_hbm, dst_hbm, z_hbm, out_hbm, sidx, didx, rb, acc_sh,
          sem0, sem1):
        c = lax.axis_index("c")
        s = lax.axis_index("s")
        w = c * NS + s

        pltpu.sync_copy(z_hbm, rb.at[0])

        def zloop(b, _):
            pltpu.sync_copy(rb.at[0], acc_sh.at[pl.ds(s * npt + b * ZB, ZB)])
            return 0

        lax.fori_loop(0, npt // ZB, zloop, 0)
        plsc.subcore_barrier()

        for p in range(nph):
            r0 = w * rpt + p * cpp
            pltpu.sync_copy(src_hbm.at[pl.ds(r0, cpp)], sidx)
            pltpu.sync_copy(dst_hbm.at[pl.ds(r0, cpp)], didx)

            if mode == "scatter":
                def body_s(j, _):
                    pltpu.sync_copy(rb.at[0], acc_sh.at[didx.at[2 * j]],
                                    add=True)
                    pltpu.sync_copy(rb.at[1], acc_sh.at[didx.at[2 * j + 1]],
                                    add=True)
                    return 0

                lax.fori_loop(0, cpp // 2, body_s, 0)
            elif mode == "gather":
                pltpu.async_copy(hs_hbm.at[sidx.at[0]], rb.at[0], sem0)

                def body_g(j, _):
                    pltpu.async_copy(hs_hbm.at[sidx.at[2 * j + 1]], rb.at[1],
                                     sem1)
                    pltpu.make_async_copy(
                        hs_hbm.at[sidx.at[2 * j]], rb.at[0], sem0).wait()

                    @pl.when(j < cpp // 2 - 1)
                    def _():
                        pltpu.async_copy(
                            hs_hbm.at[sidx.at[2 * j + 2]], rb.at[0], sem0)

                    pltpu.make_async_copy(
                        hs_hbm.at[sidx.at[2 * j + 1]], rb.at[1], sem1).wait()
                    return 0

                lax.fori_loop(0, cpp // 2, body_g, 0)
            else:
                pltpu.async_copy(hs_hbm.at[sidx.at[0]], rb.at[0], sem0)

                def body(j, _):
                    pltpu.async_copy(hs_hbm.at[sidx.at[2 * j + 1]], rb.at[1],
                                     sem1)
                    pltpu.make_async_copy(
                        hs_hbm.at[sidx.at[2 * j]], rb.at[0], sem0).wait()
                    pltpu.sync_copy(rb.at[0], acc_sh.at[didx.at[2 * j]],
                                    add=True)

                    @pl.when(j < cpp // 2 - 1)
                    def _():
                        pltpu.async_copy(
                            hs_hbm.at[sidx.at[2 * j + 2]], rb.at[0], sem0)

                    pltpu.make_async_copy(
                        hs_hbm.at[sidx.at[2 * j + 1]], rb.at[1], sem1).wait()
                    pltpu.sync_copy(rb.at[1], acc_sh.at[didx.at[2 * j + 1]],
                                    add=True)
                    return 0

                lax.fori_loop(0, cpp // 2, body, 0)

        plsc.subcore_barrier()

        def wloop(b, _):
            r0 = s * npt + b * ZB
            pltpu.sync_copy(acc_sh.at[pl.ds(r0, ZB)], rb.at[0])
            pltpu.sync_copy(rb.at[0], out_hbm.at[c, pl.ds(r0, ZB)])
            return 0

        lax.fori_loop(0, npt // ZB, wloop, 0)

    return k


def _dinv_from(degp_ref):
    deg = 1.0 + degp_ref[0, :, 0] + degp_ref[1, :, 0]
    return lax.rsqrt(deg)


def _tc_first(n, d, r):
    def body(x_ref, w_ref, degp_ref, hs_ref):
        dinv = _dinv_from(degp_ref)
        h = jnp.dot(x_ref[...], w_ref[...], preferred_element_type=jnp.float32)
        hs_ref[...] = h * dinv[:, None]

    return pl.pallas_call(
        body,
        grid=(n // r,),
        in_specs=[
            pl.BlockSpec((r, d), lambda i: (i, 0)),
            pl.BlockSpec((d, d), lambda i: (0, 0)),
            pl.BlockSpec((2, r, DEGW), lambda i: (0, i, 0)),
        ],
        out_specs=pl.BlockSpec((r, d), lambda i: (i, 0)),
        out_shape=jax.ShapeDtypeStruct((n, d), jnp.float32),
    )


def _tc_mid(n, d, r, relu):
    def body(aggp_ref, hsp_ref, b_ref, degp_ref, w_ref, out_ref):
        dinv = _dinv_from(degp_ref)
        t = (aggp_ref[0] + aggp_ref[1] + hsp_ref[...]) * dinv[:, None] + b_ref[...]
        if relu:
            t = jnp.maximum(t, 0.0)
        h = jnp.dot(t, w_ref[...], preferred_element_type=jnp.float32)
        out_ref[...] = h * dinv[:, None]

    return pl.pallas_call(
        body,
        grid=(n // r,),
        in_specs=[
            pl.BlockSpec((2, r, d), lambda i: (0, i, 0)),
            pl.BlockSpec((r, d), lambda i: (i, 0)),
            pl.BlockSpec((1, d), lambda i: (0, 0)),
            pl.BlockSpec((2, r, DEGW), lambda i: (0, i, 0)),
            pl.BlockSpec((d, d), lambda i: (0, 0)),
        ],
        out_specs=pl.BlockSpec((r, d), lambda i: (i, 0)),
        out_shape=jax.ShapeDtypeStruct((n, d), jnp.float32),
    )


def _tc_last(n, d, r):
    def body(aggp_ref, hsp_ref, degp_ref, out_ref):
        dinv = _dinv_from(degp_ref)
        out_ref[...] = (aggp_ref[0] + aggp_ref[1] + hsp_ref[...]) * dinv[:, None]

    return pl.pallas_call(
        body,
        grid=(n // r,),
        in_specs=[
            pl.BlockSpec((2, r, d), lambda i: (0, i, 0)),
            pl.BlockSpec((r, d), lambda i: (i, 0)),
            pl.BlockSpec((2, r, DEGW), lambda i: (0, i, 0)),
        ],
        out_specs=pl.BlockSpec((r, d), lambda i: (i, 0)),
        out_shape=jax.ShapeDtypeStruct((n, d), jnp.float32),
    )


def kernel(x, edge_index, W_in, b_in, W_h0, b_h0, W_h1, b_h1, W_out):
    n, d = x.shape
    e = edge_index.shape[1]
    r = 1000

    np_ = _pad_up(n + 1, NS * ZB)
    ep = _pad_up(e, NW * EK * 8)

    pad = ep - e
    pad_src = (jnp.arange(pad, dtype=edge_index.dtype) * 131) % n
    srcp = jnp.concatenate(
        [edge_index[0], pad_src]).reshape(ep // EK, EK)
    dstp = jnp.concatenate(
        [edge_index[1], jnp.full((pad,), n, edge_index.dtype)]).reshape(ep // EK, EK)

    zeros_rd = jnp.zeros((EK, d), jnp.float32)

    degp = _sc_deg(np_, ep)(dstp)
    agg_full = _sc_agg(n, np_, ep, d, "full")

    def agg(hs, sp, dp):
        return agg_full(hs, sp, dp, zeros_rd)
    first = _tc_first(n, d, r)
    mid_nr = _tc_mid(n, d, r, relu=False)
    mid_re = _tc_mid(n, d, r, relu=True)
    last = _tc_last(n, d, r)

    b2_in = b_in.reshape(1, d)
    b2_h0 = b_h0.reshape(1, d)
    b2_h1 = b_h1.reshape(1, d)

    hs0 = first(x, W_in, degp)
    a0 = agg(hs0, srcp, dstp)
    hs1 = mid_nr(a0, hs0, b2_in, degp, W_h0)
    a1 = agg(hs1, srcp, dstp)
    hs2 = mid_re(a1, hs1, b2_h0, degp, W_h1)
    a2 = agg(hs2, srcp, dstp)
    hs3 = mid_re(a2, hs2, b2_h1, degp, W_out)
    a3 = agg(hs3, srcp, dstp)
    return last(a3, hs3, degp)
